# baseline (device time: 174459 ns/iter reference)
import numpy as np

import jax
import jax.numpy as jnp
from jax import lax
from jax.experimental import pallas as pl
from jax.experimental.pallas import tpu as pltpu

N_DEV = 32
B, Sq, Skv, Hq, Dh = 2, 256, 256, 128, 64
H_LOC = Hq // N_DEV
D_MODEL = 512
ROWS = Sq // N_DEV

_qb = (np.arange(Sq) // 64)[:, None]
_kb = (np.arange(Skv) // 64)[None, :]
_MASK = (_qb == _kb) | (_kb == 0) | ((_qb + _kb) % 3 == 0)


def _allreduce_body(p_ref, out_ref, recv_buf, ss_rs, rs_rs, ss_ag, rs_ag):
    me = lax.axis_index("i")
    left = jnp.mod(me - 1, N_DEV)
    right = jnp.mod(me + 1, N_DEV)

    barrier = pltpu.get_barrier_semaphore()
    for nbr in (left, right):
        pl.semaphore_signal(
            barrier, inc=1, device_id=(nbr,),
            device_id_type=pl.DeviceIdType.MESH,
        )
    pl.semaphore_wait(barrier, 2)

    out_ref[...] = p_ref[...]

    for s in range(N_DEV - 1):
        c_send = jnp.mod(me - s, N_DEV)
        c_recv = jnp.mod(me - s - 1, N_DEV)
        rdma = pltpu.make_async_remote_copy(
            src_ref=out_ref.at[:, pl.ds(c_send * ROWS, ROWS), :],
            dst_ref=recv_buf.at[s],
            send_sem=ss_rs.at[s],
            recv_sem=rs_rs.at[s],
            device_id=(right,),
            device_id_type=pl.DeviceIdType.MESH,
        )
        rdma.start()
        rdma.wait()
        sl = pl.ds(c_recv * ROWS, ROWS)
        out_ref[:, sl, :] = out_ref[:, sl, :] + recv_buf[s]

    for s in range(N_DEV - 1):
        c = jnp.mod(me + 1 - s, N_DEV)
        sl = pl.ds(c * ROWS, ROWS)
        rdma = pltpu.make_async_remote_copy(
            src_ref=out_ref.at[:, sl, :],
            dst_ref=out_ref.at[:, sl, :],
            send_sem=ss_ag.at[s],
            recv_sem=rs_ag.at[s],
            device_id=(right,),
            device_id_type=pl.DeviceIdType.MESH,
        )
        rdma.start()
        rdma.wait()


def _ring_allreduce(p):
    return pl.pallas_call(
        _allreduce_body,
        out_shape=jax.ShapeDtypeStruct(p.shape, p.dtype),
        in_specs=[pl.BlockSpec(memory_space=pltpu.VMEM)],
        out_specs=pl.BlockSpec(memory_space=pltpu.VMEM),
        scratch_shapes=[
            pltpu.VMEM((N_DEV - 1, B, ROWS, D_MODEL), jnp.float32),
            pltpu.SemaphoreType.DMA((N_DEV - 1,)),
            pltpu.SemaphoreType.DMA((N_DEV - 1,)),
            pltpu.SemaphoreType.DMA((N_DEV - 1,)),
            pltpu.SemaphoreType.DMA((N_DEV - 1,)),
        ],
        compiler_params=pltpu.CompilerParams(collective_id=0),
    )(p)


def kernel(x, Wq, K_ext, V_ext, Wo):
    me = lax.axis_index("i")

    Q = (x @ Wq).reshape(B, Sq, H_LOC, Dh)
    K = lax.dynamic_slice_in_dim(K_ext, me * H_LOC, H_LOC, axis=2)
    V = lax.dynamic_slice_in_dim(V_ext, me * H_LOC, H_LOC, axis=2)
    scores = jnp.einsum("bihd,bjhd->bhij", Q, K) * 0.125
    scores = jnp.where(jnp.asarray(_MASK)[None, None], scores, -1e9)
    w = jax.nn.softmax(scores, axis=-1)
    ctx = jnp.einsum("bhij,bjhd->bihd", w, V).reshape(B, Sq, H_LOC * Dh)
    partial = ctx @ Wo

    return _ring_allreduce(partial)


# device time: 75233 ns/iter; 2.3189x vs baseline; 2.3189x over previous
import numpy as np

import jax
import jax.numpy as jnp
from jax import lax
from jax.experimental import pallas as pl
from jax.experimental.pallas import tpu as pltpu

N_DEV = 32
B, Sq, Skv, Hq, Dh = 2, 256, 256, 128, 64
H_LOC = Hq // N_DEV
D_MODEL = 512
ROWS = Sq // N_DEV
N_STAGES = 5

_RS_OFF = {4: 0, 3: 128, 2: 192, 1: 224, 0: 240}

_qb = (np.arange(Sq) // 64)[:, None]
_kb = (np.arange(Skv) // 64)[None, :]
_MASK = (_qb == _kb) | (_kb == 0) | ((_qb + _kb) % 3 == 0)


def _ring_to_xyz(r):
    z = r // 8
    p = r % 8
    y = p // 2
    q = p % 2
    x = jnp.where(y % 2 == 0, q, 1 - q)
    return x, y, z


def _xyz_to_ring(x, y, z):
    return z * 8 + y * 2 + jnp.where(y % 2 == 0, x, 1 - x)


def _v_to_ring(v):
    x = (v // 16) % 2
    ylo = (v // 8) % 2
    zlo = (v // 4) % 2
    yhi = (v // 2) % 2
    zhi = v % 2
    return _xyz_to_ring(x, 2 * yhi + ylo, 2 * zhi + zlo)


def _flip_bit(v, k):
    bit = (v // (1 << k)) % 2
    return v + (1 - 2 * bit) * (1 << k)


def _allreduce_body(p_ref, out_ref, recv_rs, ss_rs, rs_rs, ss_ag, rs_ag):
    me = lax.axis_index("i")
    x, y, z = _ring_to_xyz(me)
    v = x * 16 + (y % 2) * 8 + (z % 2) * 4 + (y // 2) * 2 + (z // 2)

    partners = [_v_to_ring(_flip_bit(v, k)) for k in range(N_STAGES)]

    barrier = pltpu.get_barrier_semaphore()
    for pr in partners:
        pl.semaphore_signal(
            barrier, inc=1, device_id=(pr,),
            device_id_type=pl.DeviceIdType.MESH,
        )
    pl.semaphore_wait(barrier, N_STAGES)

    out_ref[...] = p_ref[...]

    for i, k in enumerate(reversed(range(N_STAGES))):
        n = 1 << k
        base = (v // (2 * n)) * (2 * n)
        bitk = (v // n) % 2
        keep = base + bitk * n
        send = base + (1 - bitk) * n
        rdma = pltpu.make_async_remote_copy(
            src_ref=out_ref.at[:, pl.ds(send * ROWS, n * ROWS), :],
            dst_ref=recv_rs.at[:, pl.ds(_RS_OFF[k], n * ROWS), :],
            send_sem=ss_rs.at[i],
            recv_sem=rs_rs.at[i],
            device_id=(partners[k],),
            device_id_type=pl.DeviceIdType.MESH,
        )
        rdma.start()
        rdma.wait()
        sl = pl.ds(keep * ROWS, n * ROWS)
        out_ref[:, sl, :] = out_ref[:, sl, :] + recv_rs[:, pl.ds(_RS_OFF[k], n * ROWS), :]

    for k in range(N_STAGES):
        n = 1 << k
        own = (v // n) * n
        sl = pl.ds(own * ROWS, n * ROWS)
        rdma = pltpu.make_async_remote_copy(
            src_ref=out_ref.at[:, sl, :],
            dst_ref=out_ref.at[:, sl, :],
            send_sem=ss_ag.at[k],
            recv_sem=rs_ag.at[k],
            device_id=(partners[k],),
            device_id_type=pl.DeviceIdType.MESH,
        )
        rdma.start()
        rdma.wait()


def _allreduce(p):
    return pl.pallas_call(
        _allreduce_body,
        out_shape=jax.ShapeDtypeStruct(p.shape, p.dtype),
        in_specs=[pl.BlockSpec(memory_space=pltpu.VMEM)],
        out_specs=pl.BlockSpec(memory_space=pltpu.VMEM),
        scratch_shapes=[
            pltpu.VMEM((B, (N_DEV - 1) * ROWS, D_MODEL), jnp.float32),
            pltpu.SemaphoreType.DMA((N_STAGES,)),
            pltpu.SemaphoreType.DMA((N_STAGES,)),
            pltpu.SemaphoreType.DMA((N_STAGES,)),
            pltpu.SemaphoreType.DMA((N_STAGES,)),
        ],
        compiler_params=pltpu.CompilerParams(collective_id=0),
    )(p)


def kernel(x, Wq, K_ext, V_ext, Wo):
    me = lax.axis_index("i")

    Q = (x @ Wq).reshape(B, Sq, H_LOC, Dh)
    K = lax.dynamic_slice_in_dim(K_ext, me * H_LOC, H_LOC, axis=2)
    V = lax.dynamic_slice_in_dim(V_ext, me * H_LOC, H_LOC, axis=2)
    scores = jnp.einsum("bihd,bjhd->bhij", Q, K) * 0.125
    scores = jnp.where(jnp.asarray(_MASK)[None, None], scores, -1e9)
    w = jax.nn.softmax(scores, axis=-1)
    ctx = jnp.einsum("bhij,bjhd->bihd", w, V).reshape(B, Sq, H_LOC * Dh)
    partial = ctx @ Wo

    return _allreduce(partial)
